# Initial kernel scaffold; baseline (speedup 1.0000x reference)
import jax
import jax.numpy as jnp
from jax.experimental import pallas as pl


def kernel(feature, edge_index, edge_type, W_in, b_in, rgcn_weight, rgcn_root, rgcn_bias, W_out, b_out):
    raise NotImplementedError("write your pallas kernel here")


# speedup vs baseline: 3.8746x; 3.8746x over previous
import functools

import jax
import jax.numpy as jnp
from jax import lax
from jax.experimental import pallas as pl
from jax.experimental.pallas import tpu as pltpu
from jax.experimental.pallas import tpu_sc as plsc

N = 10000
E = 320000
D = 128
H = 32
NREL = 2
NC = 2
NS = 16
L = 16
B = 128
NBT = 160
EW = NBT * B
E_PAD = NS * EW
NROWS = 20480
RPTA = NROWS // NS
NRC = 10240
RPTC = NRC // NS
BN = 1000
GRID = N // BN

_mesh = functools.lru_cache(maxsize=None)(
    lambda: plsc.VectorSubcoreMesh(core_axis_name="c", subcore_axis_name="s",
                                   num_cores=NC, num_subcores=NS))

_sc_params = pltpu.CompilerParams(use_tc_tiling_on_sc=False,
                                  needs_layout_passes=False,
                                  internal_scratch_in_bytes=2 * 1024 * 1024)


ERPT = EW // H
EROW0 = 4 * N


def _load_edges(edges_pr, pck_v, s, sem, base, npr):
    iota = lax.iota(jnp.int32, L)
    for k in range(npr // L):
        idx = iota + (base + k * L)
        pltpu.async_copy(edges_pr.at[idx], pck_v.at[pl.ds(k * L, L)],
                         sem).wait()


@functools.lru_cache(maxsize=None)
def _make_conv_sc(k: int):
    scratch_types = [
        pltpu.VMEM((NBT, B), jnp.int32),
        pltpu.VMEM((NBT, B), jnp.int32),
        pltpu.VMEM((NBT, B), jnp.int32),
        pltpu.VMEM((B, H), jnp.float32),
        pltpu.VMEM((B, H), jnp.float32),
        pltpu.VMEM_SHARED((NROWS, H), jnp.float32),
        pltpu.SemaphoreType.DMA,
        pltpu.SemaphoreType.DMA,
    ]

    def body(xg, edges_pr, agg_out, pck_v, src_v, key_v, rows_v, zH_v,
             acc_sh, sem_g, sem_s):
        c = lax.axis_index("c")
        s = lax.axis_index("s")

        zH = jnp.zeros((L,), jnp.float32)

        def init_row(i, _):
            for jl in range(H // L):
                zH_v[i, pl.ds(jl * L, L)] = zH
            return 0
        lax.fori_loop(0, B, init_row, 0)

        for t in range(RPTA // B):
            pltpu.sync_copy(zH_v, acc_sh.at[pl.ds(s * RPTA + t * B, B)])

        _load_edges(edges_pr, pck_v, s, sem_g, s * NBT, NBT)

        qvec = jnp.full((L,), 2 * k, jnp.int32) + c
        msk = jnp.full((L,), 0x3FFF, jnp.int32)

        def idx_body(jb, _):
            for jl in range(B // L):
                sl = pl.ds(jl * L, L)
                p = pck_v[jb, sl]
                src_v[jb, sl] = ((p & msk) << 2) + qvec
                key_v[jb, sl] = ((p >> 28) & 3) * N + ((p >> 14) & msk)
            return 0
        lax.fori_loop(0, NBT, idx_body, 0)

        plsc.subcore_barrier()

        def edge_body(jb, _):
            pltpu.async_copy(xg.at[src_v.at[jb]], rows_v, sem_g).wait()
            pltpu.async_copy(rows_v, acc_sh.at[key_v.at[jb]], sem_s,
                             add=True).wait()
            return 0
        lax.fori_loop(0, NBT, edge_body, 0)

        plsc.subcore_barrier()

        for t in range(RPTA // B):
            r0 = s * RPTA + t * B
            pltpu.sync_copy(acc_sh.at[pl.ds(r0, B)],
                            agg_out.at[c, pl.ds(r0, B)])

    return pl.kernel(body,
                     out_type=jax.ShapeDtypeStruct((NC, NROWS, H),
                                                   jnp.float32),
                     mesh=_mesh(), scratch_types=scratch_types,
                     compiler_params=_sc_params,
                     name=f"rgcn_edge_sc{k}")


@functools.lru_cache(maxsize=None)
def _make_count_sc():
    scratch_types = [
        pltpu.VMEM((NBT, B), jnp.int32),
        pltpu.VMEM((NBT, B), jnp.int32),
        pltpu.VMEM((B, L), jnp.float32),
        pltpu.VMEM((B, L), jnp.float32),
        pltpu.VMEM_SHARED((NRC, L), jnp.float32),
        pltpu.SemaphoreType.DMA,
    ]

    def body(edges_pr, cnt_out, pck_v, key_v, zL_v, ones_v, cnt_sh, sem_c):
        c = lax.axis_index("c")
        s = lax.axis_index("s")

        zL = jnp.zeros((L,), jnp.float32)
        one = jnp.full((L,), 1.0, jnp.float32)

        def init_row(i, _):
            zL_v[i] = zL
            ones_v[i] = one
            return 0
        lax.fori_loop(0, B, init_row, 0)

        for t in range(RPTC // B):
            pltpu.sync_copy(zL_v, cnt_sh.at[pl.ds(s * RPTC + t * B, B)])

        _load_edges(edges_pr, pck_v, s, sem_c, s * NBT, NBT)

        msk = jnp.full((L,), 0x3FFF, jnp.int32)
        dummy = jnp.full((L,), N, jnp.int32)

        def idx_body(jb, _):
            for jl in range(B // L):
                sl = pl.ds(jl * L, L)
                p = pck_v[jb, sl]
                typ = (p >> 28) & 3
                key_v[jb, sl] = jnp.where(typ == c, (p >> 14) & msk, dummy)
            return 0
        lax.fori_loop(0, NBT, idx_body, 0)

        plsc.subcore_barrier()

        def edge_body(jb, _):
            pltpu.async_copy(ones_v, cnt_sh.at[key_v.at[jb]], sem_c,
                             add=True).wait()
            return 0
        lax.fori_loop(0, NBT, edge_body, 0)

        plsc.subcore_barrier()

        for t in range(RPTC // B):
            r0 = s * RPTC + t * B
            pltpu.sync_copy(cnt_sh.at[pl.ds(r0, B)],
                            cnt_out.at[c, pl.ds(r0, B)])

    return pl.kernel(body,
                     out_type=jax.ShapeDtypeStruct((NC, NRC, L), jnp.float32),
                     mesh=_mesh(), scratch_types=scratch_types,
                     compiler_params=_sc_params,
                     name="rgcn_count_sc")


def _proj_body(f_ref, w_ref, b_ref, o_ref):
    h = jnp.dot(f_ref[...], w_ref[...], preferred_element_type=jnp.float32)
    h = h + b_ref[...]
    o_ref[...] = jnp.where(h >= 0, h, 0.01 * h)


def _proj(feature, W_in, b_in2):
    return pl.pallas_call(
        _proj_body,
        grid=(GRID,),
        in_specs=[
            pl.BlockSpec((BN, D), lambda i: (i, 0)),
            pl.BlockSpec((D, D), lambda i: (0, 0)),
            pl.BlockSpec((1, D), lambda i: (0, 0)),
        ],
        out_specs=pl.BlockSpec((BN, D), lambda i: (i, 0)),
        out_shape=jax.ShapeDtypeStruct((N, D), jnp.float32),
    )(feature, W_in, b_in2)


def _combine_body(final, aA0_ref, aA1_ref, aB0_ref, aB1_ref, c0_ref, c1_ref,
                  x_ref, w_ref, root_ref, bias_ref, *rest):
    if final:
        wo_ref, bo_ref, o_ref = rest
    else:
        (o_ref,) = rest
    inv0 = 1.0 / jnp.maximum(c0_ref[0, :, 0:1], 1.0)
    inv1 = 1.0 / jnp.maximum(c1_ref[0, :, 0:1], 1.0)
    f32 = jnp.float32
    h = jnp.dot(x_ref[...], root_ref[...], preferred_element_type=f32)
    for r, (a_ref, b_ref, inv) in enumerate(
            [(aA0_ref, aB0_ref, inv0), (aA1_ref, aB1_ref, inv1)]):
        for q in range(4):
            ref = a_ref if q < 2 else b_ref
            h += jnp.dot(ref[q % 2] * inv, w_ref[r, q * H:(q + 1) * H, :],
                         preferred_element_type=f32)
    h += bias_ref[...]
    if final:
        o_ref[...] = jnp.dot(h, wo_ref[...], preferred_element_type=f32) \
            + bo_ref[...]
    else:
        o_ref[...] = h


def _combine(final, aggA, aggB, cnt, x, rgcn_weight, rgcn_root, bias2, *wout):
    rel1 = N // BN
    in_specs = [
        pl.BlockSpec((NC, BN, H), lambda i: (0, i, 0)),
        pl.BlockSpec((NC, BN, H), lambda i: (0, rel1 + i, 0)),
        pl.BlockSpec((NC, BN, H), lambda i: (0, i, 0)),
        pl.BlockSpec((NC, BN, H), lambda i: (0, rel1 + i, 0)),
        pl.BlockSpec((1, BN, L), lambda i: (0, i, 0)),
        pl.BlockSpec((1, BN, L), lambda i: (1, i, 0)),
        pl.BlockSpec((BN, D), lambda i: (i, 0)),
        pl.BlockSpec((NREL, D, D), lambda i: (0, 0, 0)),
        pl.BlockSpec((D, D), lambda i: (0, 0)),
        pl.BlockSpec((1, D), lambda i: (0, 0)),
    ]
    args = [aggA, aggA, aggB, aggB, cnt, cnt, x, rgcn_weight, rgcn_root,
            bias2]
    if final:
        in_specs += [pl.BlockSpec((D, D), lambda i: (0, 0)),
                     pl.BlockSpec((1, D), lambda i: (0, 0))]
        args += list(wout)
    return pl.pallas_call(
        functools.partial(_combine_body, final),
        grid=(GRID,),
        in_specs=in_specs,
        out_specs=pl.BlockSpec((BN, D), lambda i: (i, 0)),
        out_shape=jax.ShapeDtypeStruct((N, D), jnp.float32),
    )(*args)


def kernel(feature, edge_index, edge_type, W_in, b_in, rgcn_weight,
           rgcn_root, rgcn_bias, W_out, b_out):
    f32 = jnp.float32
    pad = E_PAD - E
    packed = (edge_index[0] | (edge_index[1] << 14)
              | (edge_type << 28)).astype(jnp.int32)
    edges_pr = jnp.concatenate(
        [packed, jnp.full((pad,), NREL << 28, jnp.int32)]).reshape(
            NS * NBT, B)

    b_in2 = b_in.reshape(1, D).astype(f32)
    bias2 = rgcn_bias.reshape(1, D).astype(f32)
    wo_pad = jnp.zeros((D, D), f32).at[:, : W_out.shape[1]].set(W_out)
    bo_pad = jnp.zeros((1, D), f32).at[0, : W_out.shape[1]].set(b_out)

    cnt = _make_count_sc()(edges_pr)
    x = _proj(feature.astype(f32), W_in, b_in2)
    agg1A = _make_conv_sc(0)(x.reshape(4 * N, H), edges_pr)
    agg1B = _make_conv_sc(1)(x.reshape(4 * N, H), edges_pr)
    x2 = _combine(False, agg1A, agg1B, cnt, x, rgcn_weight, rgcn_root, bias2)
    agg2A = _make_conv_sc(0)(x2.reshape(4 * N, H), edges_pr)
    agg2B = _make_conv_sc(1)(x2.reshape(4 * N, H), edges_pr)
    out = _combine(True, agg2A, agg2B, cnt, x2, rgcn_weight, rgcn_root,
                   bias2, wo_pad, bo_pad)
    return out[:, : W_out.shape[1]]


# speedup vs baseline: 4.1293x; 1.0657x over previous
import functools

import jax
import jax.numpy as jnp
from jax import lax
from jax.experimental import pallas as pl
from jax.experimental.pallas import tpu as pltpu
from jax.experimental.pallas import tpu_sc as plsc

N = 10000
E = 320000
D = 128
H = 32
NREL = 2
NC = 2
NS = 16
L = 16
B = 128
NBT = 160
EW = NBT * B
E_PAD = NS * EW
NROWS = 20480
RPTA = NROWS // NS
NRC = 10240
RPTC = NRC // NS
BN = 1000
GRID = N // BN

_mesh = functools.lru_cache(maxsize=None)(
    lambda: plsc.VectorSubcoreMesh(core_axis_name="c", subcore_axis_name="s",
                                   num_cores=NC, num_subcores=NS))

_sc_params = pltpu.CompilerParams(use_tc_tiling_on_sc=False,
                                  needs_layout_passes=False,
                                  internal_scratch_in_bytes=2 * 1024 * 1024)


ERPT = EW // H
EROW0 = 4 * N


def _load_edges(edges_pr, pck_v, s, sem, base, npr):
    iota = lax.iota(jnp.int32, L)
    for k in range(npr // L):
        idx = iota + (base + k * L)
        pltpu.async_copy(edges_pr.at[idx], pck_v.at[pl.ds(k * L, L)],
                         sem).wait()


@functools.lru_cache(maxsize=None)
def _make_conv_sc(k: int):
    scratch_types = [
        pltpu.VMEM((NBT, B), jnp.int32),
        pltpu.VMEM((NBT, B), jnp.int32),
        pltpu.VMEM((NBT, B), jnp.int32),
        pltpu.VMEM((B, H), jnp.float32),
        pltpu.VMEM((B, H), jnp.float32),
        pltpu.VMEM((B, H), jnp.float32),
        pltpu.VMEM_SHARED((NROWS, H), jnp.float32),
        pltpu.SemaphoreType.DMA,
        pltpu.SemaphoreType.DMA,
        pltpu.SemaphoreType.DMA,
        pltpu.SemaphoreType.DMA,
    ]

    def body(xg, edges_pr, agg_out, pck_v, src_v, key_v, rows0_v, rows1_v,
             zH_v, acc_sh, sem_g0, sem_g1, sem_s0, sem_s1):
        c = lax.axis_index("c")
        s = lax.axis_index("s")

        zH = jnp.zeros((L,), jnp.float32)

        def init_row(i, _):
            for jl in range(H // L):
                zH_v[i, pl.ds(jl * L, L)] = zH
            return 0
        lax.fori_loop(0, B, init_row, 0)

        for t in range(RPTA // B):
            pltpu.sync_copy(zH_v, acc_sh.at[pl.ds(s * RPTA + t * B, B)])

        _load_edges(edges_pr, pck_v, s, sem_g0, s * NBT, NBT)

        qvec = jnp.full((L,), 2 * k, jnp.int32) + c
        msk = jnp.full((L,), 0x3FFF, jnp.int32)

        def idx_body(jb, _):
            for jl in range(B // L):
                sl = pl.ds(jl * L, L)
                p = pck_v[jb, sl]
                src_v[jb, sl] = ((p & msk) << 2) + qvec
                key_v[jb, sl] = ((p >> 28) & 3) * N + ((p >> 14) & msk)
            return 0
        lax.fori_loop(0, NBT, idx_body, 0)

        plsc.subcore_barrier()

        def g_start(jb, buf, sem):
            pltpu.async_copy(xg.at[src_v.at[jb]], buf, sem)

        def g_wait(jb, buf, sem):
            pltpu.make_async_copy(xg.at[src_v.at[jb]], buf, sem).wait()

        def s_start(jb, buf, sem):
            pltpu.async_copy(buf, acc_sh.at[key_v.at[jb]], sem, add=True)

        def s_wait(jb, buf, sem):
            pltpu.make_async_copy(buf, acc_sh.at[key_v.at[jb]], sem).wait()

        g_start(0, rows0_v, sem_g0)

        def pair_body(jj, _):
            jb0 = 2 * jj
            jb1 = jb0 + 1
            g_wait(jb0, rows0_v, sem_g0)

            @pl.when(jj > 0)
            def _():
                s_wait(jb0, rows1_v, sem_s1)

            g_start(jb1, rows1_v, sem_g1)
            s_start(jb0, rows0_v, sem_s0)
            g_wait(jb1, rows1_v, sem_g1)
            s_wait(jb0, rows0_v, sem_s0)

            @pl.when(jj < NBT // 2 - 1)
            def _():
                g_start(jb0 + 2, rows0_v, sem_g0)

            s_start(jb1, rows1_v, sem_s1)
            return 0
        lax.fori_loop(0, NBT // 2, pair_body, 0)
        s_wait(NBT - 1, rows1_v, sem_s1)

        plsc.subcore_barrier()

        for t in range(RPTA // B):
            r0 = s * RPTA + t * B
            pltpu.sync_copy(acc_sh.at[pl.ds(r0, B)],
                            agg_out.at[c, pl.ds(r0, B)])

    return pl.kernel(body,
                     out_type=jax.ShapeDtypeStruct((NC, NROWS, H),
                                                   jnp.float32),
                     mesh=_mesh(), scratch_types=scratch_types,
                     compiler_params=_sc_params,
                     name=f"rgcn_edge_sc{k}")


@functools.lru_cache(maxsize=None)
def _make_count_sc():
    scratch_types = [
        pltpu.VMEM((NBT, B), jnp.int32),
        pltpu.VMEM((NBT, B), jnp.int32),
        pltpu.VMEM((B, L), jnp.float32),
        pltpu.VMEM((B, L), jnp.float32),
        pltpu.VMEM_SHARED((NRC, L), jnp.float32),
        pltpu.SemaphoreType.DMA,
    ]

    def body(edges_pr, cnt_out, pck_v, key_v, zL_v, ones_v, cnt_sh, sem_c):
        c = lax.axis_index("c")
        s = lax.axis_index("s")

        zL = jnp.zeros((L,), jnp.float32)
        one = jnp.full((L,), 1.0, jnp.float32)

        def init_row(i, _):
            zL_v[i] = zL
            ones_v[i] = one
            return 0
        lax.fori_loop(0, B, init_row, 0)

        for t in range(RPTC // B):
            pltpu.sync_copy(zL_v, cnt_sh.at[pl.ds(s * RPTC + t * B, B)])

        _load_edges(edges_pr, pck_v, s, sem_c, s * NBT, NBT)

        msk = jnp.full((L,), 0x3FFF, jnp.int32)
        dummy = jnp.full((L,), N, jnp.int32)

        def idx_body(jb, _):
            for jl in range(B // L):
                sl = pl.ds(jl * L, L)
                p = pck_v[jb, sl]
                typ = (p >> 28) & 3
                key_v[jb, sl] = jnp.where(typ == c, (p >> 14) & msk, dummy)
            return 0
        lax.fori_loop(0, NBT, idx_body, 0)

        plsc.subcore_barrier()

        def edge_body(jb, _):
            pltpu.async_copy(ones_v, cnt_sh.at[key_v.at[jb]], sem_c,
                             add=True).wait()
            return 0
        lax.fori_loop(0, NBT, edge_body, 0)

        plsc.subcore_barrier()

        for t in range(RPTC // B):
            r0 = s * RPTC + t * B
            pltpu.sync_copy(cnt_sh.at[pl.ds(r0, B)],
                            cnt_out.at[c, pl.ds(r0, B)])

    return pl.kernel(body,
                     out_type=jax.ShapeDtypeStruct((NC, NRC, L), jnp.float32),
                     mesh=_mesh(), scratch_types=scratch_types,
                     compiler_params=_sc_params,
                     name="rgcn_count_sc")


def _proj_body(f_ref, w_ref, b_ref, o_ref):
    h = jnp.dot(f_ref[...], w_ref[...], preferred_element_type=jnp.float32)
    h = h + b_ref[...]
    o_ref[...] = jnp.where(h >= 0, h, 0.01 * h)


def _proj(feature, W_in, b_in2):
    return pl.pallas_call(
        _proj_body,
        grid=(GRID,),
        in_specs=[
            pl.BlockSpec((BN, D), lambda i: (i, 0)),
            pl.BlockSpec((D, D), lambda i: (0, 0)),
            pl.BlockSpec((1, D), lambda i: (0, 0)),
        ],
        out_specs=pl.BlockSpec((BN, D), lambda i: (i, 0)),
        out_shape=jax.ShapeDtypeStruct((N, D), jnp.float32),
    )(feature, W_in, b_in2)


def _combine_body(final, aA0_ref, aA1_ref, aB0_ref, aB1_ref, c0_ref, c1_ref,
                  x_ref, w_ref, root_ref, bias_ref, *rest):
    if final:
        wo_ref, bo_ref, o_ref = rest
    else:
        (o_ref,) = rest
    inv0 = 1.0 / jnp.maximum(c0_ref[0, :, 0:1], 1.0)
    inv1 = 1.0 / jnp.maximum(c1_ref[0, :, 0:1], 1.0)
    f32 = jnp.float32
    h = jnp.dot(x_ref[...], root_ref[...], preferred_element_type=f32)
    for r, (a_ref, b_ref, inv) in enumerate(
            [(aA0_ref, aB0_ref, inv0), (aA1_ref, aB1_ref, inv1)]):
        for q in range(4):
            ref = a_ref if q < 2 else b_ref
            h += jnp.dot(ref[q % 2] * inv, w_ref[r, q * H:(q + 1) * H, :],
                         preferred_element_type=f32)
    h += bias_ref[...]
    if final:
        o_ref[...] = jnp.dot(h, wo_ref[...], preferred_element_type=f32) \
            + bo_ref[...]
    else:
        o_ref[...] = h


def _combine(final, aggA, aggB, cnt, x, rgcn_weight, rgcn_root, bias2, *wout):
    rel1 = N // BN
    in_specs = [
        pl.BlockSpec((NC, BN, H), lambda i: (0, i, 0)),
        pl.BlockSpec((NC, BN, H), lambda i: (0, rel1 + i, 0)),
        pl.BlockSpec((NC, BN, H), lambda i: (0, i, 0)),
        pl.BlockSpec((NC, BN, H), lambda i: (0, rel1 + i, 0)),
        pl.BlockSpec((1, BN, L), lambda i: (0, i, 0)),
        pl.BlockSpec((1, BN, L), lambda i: (1, i, 0)),
        pl.BlockSpec((BN, D), lambda i: (i, 0)),
        pl.BlockSpec((NREL, D, D), lambda i: (0, 0, 0)),
        pl.BlockSpec((D, D), lambda i: (0, 0)),
        pl.BlockSpec((1, D), lambda i: (0, 0)),
    ]
    args = [aggA, aggA, aggB, aggB, cnt, cnt, x, rgcn_weight, rgcn_root,
            bias2]
    if final:
        in_specs += [pl.BlockSpec((D, D), lambda i: (0, 0)),
                     pl.BlockSpec((1, D), lambda i: (0, 0))]
        args += list(wout)
    return pl.pallas_call(
        functools.partial(_combine_body, final),
        grid=(GRID,),
        in_specs=in_specs,
        out_specs=pl.BlockSpec((BN, D), lambda i: (i, 0)),
        out_shape=jax.ShapeDtypeStruct((N, D), jnp.float32),
    )(*args)


def kernel(feature, edge_index, edge_type, W_in, b_in, rgcn_weight,
           rgcn_root, rgcn_bias, W_out, b_out):
    f32 = jnp.float32
    pad = E_PAD - E
    packed = (edge_index[0] | (edge_index[1] << 14)
              | (edge_type << 28)).astype(jnp.int32)
    edges_pr = jnp.concatenate(
        [packed, jnp.full((pad,), NREL << 28, jnp.int32)]).reshape(
            NS * NBT, B)

    b_in2 = b_in.reshape(1, D).astype(f32)
    bias2 = rgcn_bias.reshape(1, D).astype(f32)
    wo_pad = jnp.zeros((D, D), f32).at[:, : W_out.shape[1]].set(W_out)
    bo_pad = jnp.zeros((1, D), f32).at[0, : W_out.shape[1]].set(b_out)

    cnt = _make_count_sc()(edges_pr)
    x = _proj(feature.astype(f32), W_in, b_in2)
    agg1A = _make_conv_sc(0)(x.reshape(4 * N, H), edges_pr)
    agg1B = _make_conv_sc(1)(x.reshape(4 * N, H), edges_pr)
    x2 = _combine(False, agg1A, agg1B, cnt, x, rgcn_weight, rgcn_root, bias2)
    agg2A = _make_conv_sc(0)(x2.reshape(4 * N, H), edges_pr)
    agg2B = _make_conv_sc(1)(x2.reshape(4 * N, H), edges_pr)
    out = _combine(True, agg2A, agg2B, cnt, x2, rgcn_weight, rgcn_root,
                   bias2, wo_pad, bo_pad)
    return out[:, : W_out.shape[1]]


# speedup vs baseline: 4.1412x; 1.0029x over previous
import functools

import jax
import jax.numpy as jnp
from jax import lax
from jax.experimental import pallas as pl
from jax.experimental.pallas import tpu as pltpu
from jax.experimental.pallas import tpu_sc as plsc

N = 10000
E = 320000
D = 128
H = 64
NREL = 2
NC = 2
NS = 16
L = 16
B = 128
NBT = 160
NBH = NBT // 4
PCKB = ((NBH + L - 1) // L) * L
EW = NBT * B
E_PAD = NS * EW
NROWS = 20480
RPTA = NROWS // NS
NRC = 10240
RPTC = NRC // NS
BN = 1000
GRID = N // BN

_mesh = functools.lru_cache(maxsize=None)(
    lambda: plsc.VectorSubcoreMesh(core_axis_name="c", subcore_axis_name="s",
                                   num_cores=NC, num_subcores=NS))

_sc_params = pltpu.CompilerParams(use_tc_tiling_on_sc=False,
                                  needs_layout_passes=False)


def _load_edges(edges_h, pck_v, s, sem, base, npr, nrows):
    iota = lax.iota(jnp.int32, L)
    for k in range((npr + L - 1) // L):
        idx = jnp.minimum(iota + (base + k * L),
                          jnp.full((L,), nrows - 1, jnp.int32))
        pltpu.async_copy(edges_h.at[idx], pck_v.at[pl.ds(k * L, L)],
                         sem).wait()


@functools.lru_cache(maxsize=None)
def _make_conv_sc(with_counts: bool):
    scratch_types = [
        pltpu.VMEM((PCKB, B), jnp.int32),
        pltpu.VMEM((NBH, B), jnp.int32),
        pltpu.VMEM((NBH, B), jnp.int32),
        pltpu.VMEM((B, H), jnp.float32),
        pltpu.VMEM((B, H), jnp.float32),
        pltpu.VMEM((B, H), jnp.float32),
        pltpu.VMEM_SHARED((NROWS, H), jnp.float32),
        pltpu.SemaphoreType.DMA,
        pltpu.SemaphoreType.DMA,
        pltpu.SemaphoreType.DMA,
        pltpu.SemaphoreType.DMA,
    ]
    if with_counts:
        scratch_types += [
            pltpu.VMEM((NBH, B), jnp.int32),
            pltpu.VMEM((B, L), jnp.float32),
            pltpu.VMEM((B, L), jnp.float32),
            pltpu.VMEM_SHARED((NRC, L), jnp.float32),
            pltpu.SemaphoreType.DMA,
        ]

    def body(xg, edges_h, *rest):
        if with_counts:
            agg_out, cnt_out = rest[0], rest[1]
            (pck_v, src_v, key_v, rows0_v, rows1_v, zH_v, acc_sh,
             sem_g0, sem_g1, sem_s0, sem_s1,
             key2_v, zL_v, ones_v, cnt_sh, sem_c) = rest[2:]
        else:
            agg_out = rest[0]
            (pck_v, src_v, key_v, rows0_v, rows1_v, zH_v, acc_sh,
             sem_g0, sem_g1, sem_s0, sem_s1) = rest[1:]

        c = lax.axis_index("c")
        s = lax.axis_index("s")

        zH = jnp.zeros((L,), jnp.float32)

        def init_row(i, _):
            for jl in range(H // L):
                zH_v[i, pl.ds(jl * L, L)] = zH
            if with_counts:
                zL_v[i] = zH
                ones_v[i] = jnp.full((L,), 1.0, jnp.float32)
            return 0
        lax.fori_loop(0, B, init_row, 0)

        for t in range(RPTA // B):
            pltpu.sync_copy(zH_v, acc_sh.at[pl.ds(s * RPTA + t * B, B)])
        if with_counts:
            for t in range(RPTC // B):
                pltpu.sync_copy(zL_v, cnt_sh.at[pl.ds(s * RPTC + t * B, B)])

        _load_edges(edges_h, pck_v, s, sem_g0, s * NBH, NBH, NS * NBH)

        cvec = jnp.full((L,), c, jnp.int32)
        msk = jnp.full((L,), 0x3FFF, jnp.int32)
        dummy = jnp.full((L,), N, jnp.int32)

        def idx_body(jb, _):
            for jl in range(B // L):
                sl = pl.ds(jl * L, L)
                p = pck_v[jb, sl]
                typ = (p >> 28) & 3
                dst = (p >> 14) & msk
                src_v[jb, sl] = ((p & msk) << 1) + cvec
                key_v[jb, sl] = typ * N + dst
                if with_counts:
                    key2_v[jb, sl] = jnp.where(typ == c, dst, dummy)
            return 0
        lax.fori_loop(0, NBH, idx_body, 0)

        plsc.subcore_barrier()

        def g_start(jb, buf, sem):
            pltpu.async_copy(xg.at[src_v.at[jb]], buf, sem)

        def g_wait(jb, buf, sem):
            pltpu.make_async_copy(xg.at[src_v.at[jb]], buf, sem).wait()

        def s_start(jb, buf, sem):
            pltpu.async_copy(buf, acc_sh.at[key_v.at[jb]], sem, add=True)

        def s_wait(jb, buf, sem):
            pltpu.make_async_copy(buf, acc_sh.at[key_v.at[jb]], sem).wait()

        def c_start(jb):
            pltpu.async_copy(ones_v, cnt_sh.at[key2_v.at[jb]], sem_c,
                             add=True)

        def c_wait(jb):
            pltpu.make_async_copy(ones_v, cnt_sh.at[key2_v.at[jb]],
                                  sem_c).wait()

        g_start(0, rows0_v, sem_g0)

        def pair_body(jj, _):
            jb0 = 2 * jj
            jb1 = jb0 + 1
            g_wait(jb0, rows0_v, sem_g0)

            @pl.when(jj > 0)
            def _():
                s_wait(jb0, rows1_v, sem_s1)
                if with_counts:
                    c_wait(jb0)
                    c_wait(jb0)

            g_start(jb1, rows1_v, sem_g1)
            s_start(jb0, rows0_v, sem_s0)
            if with_counts:
                c_start(jb0)
            g_wait(jb1, rows1_v, sem_g1)
            s_wait(jb0, rows0_v, sem_s0)

            @pl.when(jj < NBH // 2 - 1)
            def _():
                g_start(jb0 + 2, rows0_v, sem_g0)

            s_start(jb1, rows1_v, sem_s1)
            if with_counts:
                c_start(jb1)
            return 0
        lax.fori_loop(0, NBH // 2, pair_body, 0)
        s_wait(NBH - 1, rows1_v, sem_s1)
        if with_counts:
            c_wait(0)
            c_wait(0)

        plsc.subcore_barrier()

        for t in range(RPTA // B):
            r0 = s * RPTA + t * B
            pltpu.sync_copy(acc_sh.at[pl.ds(r0, B)],
                            agg_out.at[c, pl.ds(r0, B)])
        if with_counts:
            for t in range(RPTC // B):
                r0 = s * RPTC + t * B
                pltpu.sync_copy(cnt_sh.at[pl.ds(r0, B)],
                                cnt_out.at[c, pl.ds(r0, B)])

    out_type = [jax.ShapeDtypeStruct((NC, NROWS, H), jnp.float32)]
    if with_counts:
        out_type.append(jax.ShapeDtypeStruct((NC, NRC, L), jnp.float32))
    return pl.kernel(body, out_type=tuple(out_type),
                     mesh=_mesh(), scratch_types=scratch_types,
                     compiler_params=_sc_params,
                     name=f"rgcn_edge_sc{'_cnt' if with_counts else ''}")


@functools.lru_cache(maxsize=None)
def _make_count_sc():
    scratch_types = [
        pltpu.VMEM((NBT, B), jnp.int32),
        pltpu.VMEM((NBT, B), jnp.int32),
        pltpu.VMEM((B, L), jnp.float32),
        pltpu.VMEM((B, L), jnp.float32),
        pltpu.VMEM_SHARED((NRC, L), jnp.float32),
        pltpu.SemaphoreType.DMA,
    ]
    DEPTH = 2

    def body(edges_pr, cnt_out, pck_v, key_v, zL_v, ones_v, cnt_sh, sem_c):
        c = lax.axis_index("c")
        s = lax.axis_index("s")

        zL = jnp.zeros((L,), jnp.float32)
        one = jnp.full((L,), 1.0, jnp.float32)

        def init_row(i, _):
            zL_v[i] = zL
            ones_v[i] = one
            return 0
        lax.fori_loop(0, B, init_row, 0)

        for t in range(RPTC // B):
            pltpu.sync_copy(zL_v, cnt_sh.at[pl.ds(s * RPTC + t * B, B)])

        _load_edges(edges_pr, pck_v, s, sem_c, s * NBT, NBT, NS * NBT)

        msk = jnp.full((L,), 0x3FFF, jnp.int32)
        dummy = jnp.full((L,), N, jnp.int32)

        def idx_body(jb, _):
            for jl in range(B // L):
                sl = pl.ds(jl * L, L)
                p = pck_v[jb, sl]
                key_v[jb, sl] = jnp.where(((p >> 28) & 3) == c,
                                          (p >> 14) & msk, dummy)
            return 0
        lax.fori_loop(0, NBT, idx_body, 0)

        plsc.subcore_barrier()

        def c_start(jb):
            pltpu.async_copy(ones_v, cnt_sh.at[key_v.at[jb]], sem_c,
                             add=True)

        def c_wait(jb):
            pltpu.make_async_copy(ones_v, cnt_sh.at[key_v.at[jb]],
                                  sem_c).wait()

        def edge_body(jb, _):
            @pl.when(jb >= DEPTH)
            def _():
                c_wait(jb)
            c_start(jb)
            return 0
        lax.fori_loop(0, NBT, edge_body, 0)
        for _ in range(DEPTH):
            c_wait(0)

        plsc.subcore_barrier()

        for t in range(RPTC // B):
            r0 = s * RPTC + t * B
            pltpu.sync_copy(cnt_sh.at[pl.ds(r0, B)],
                            cnt_out.at[c, pl.ds(r0, B)])

    return pl.kernel(body,
                     out_type=jax.ShapeDtypeStruct((NC, NRC, L), jnp.float32),
                     mesh=_mesh(), scratch_types=scratch_types,
                     compiler_params=_sc_params,
                     name="rgcn_count_sc")


def _proj_body(f_ref, w_ref, b_ref, o_ref):
    h = jnp.dot(f_ref[...], w_ref[...], preferred_element_type=jnp.float32)
    h = h + b_ref[...]
    o_ref[...] = jnp.where(h >= 0, h, 0.01 * h)


def _proj(feature, W_in, b_in2):
    return pl.pallas_call(
        _proj_body,
        grid=(GRID,),
        in_specs=[
            pl.BlockSpec((BN, D), lambda i: (i, 0)),
            pl.BlockSpec((D, D), lambda i: (0, 0)),
            pl.BlockSpec((1, D), lambda i: (0, 0)),
        ],
        out_specs=pl.BlockSpec((BN, D), lambda i: (i, 0)),
        out_shape=jax.ShapeDtypeStruct((N, D), jnp.float32),
    )(feature, W_in, b_in2)


NPART = 4


def _combine_body(final, *refs):
    a_refs = refs[: 2 * NPART]
    (c0_ref, c1_ref, x_ref, w_ref, root_ref, bias_ref), rest = \
        refs[2 * NPART: 2 * NPART + 6], refs[2 * NPART + 6:]
    if final:
        wo_ref, bo_ref, o_ref = rest
    else:
        (o_ref,) = rest
    inv0 = 1.0 / jnp.maximum(c0_ref[0, :, 0:1], 1.0)
    inv1 = 1.0 / jnp.maximum(c1_ref[0, :, 0:1], 1.0)
    f32 = jnp.float32
    h = jnp.dot(x_ref[...], root_ref[...], preferred_element_type=f32)
    for r, inv in [(0, inv0), (1, inv1)]:
        for cidx in range(2):
            agg = a_refs[r][cidx]
            for pp in range(1, NPART):
                agg = agg + a_refs[2 * pp + r][cidx]
            h += jnp.dot(agg * inv, w_ref[r, cidx * H:(cidx + 1) * H, :],
                         preferred_element_type=f32)
    h += bias_ref[...]
    if final:
        o_ref[...] = jnp.dot(h, wo_ref[...], preferred_element_type=f32) \
            + bo_ref[...]
    else:
        o_ref[...] = h


def _combine(final, aggs, cnt, x, rgcn_weight, rgcn_root, bias2, *wout):
    rel1 = N // BN
    in_specs = []
    args = []
    for a in aggs:
        in_specs += [pl.BlockSpec((NC, BN, H), lambda i: (0, i, 0)),
                     pl.BlockSpec((NC, BN, H), lambda i: (0, rel1 + i, 0))]
        args += [a, a]
    in_specs += [
        pl.BlockSpec((1, BN, L), lambda i: (0, i, 0)),
        pl.BlockSpec((1, BN, L), lambda i: (1, i, 0)),
        pl.BlockSpec((BN, D), lambda i: (i, 0)),
        pl.BlockSpec((NREL, D, D), lambda i: (0, 0, 0)),
        pl.BlockSpec((D, D), lambda i: (0, 0)),
        pl.BlockSpec((1, D), lambda i: (0, 0)),
    ]
    args += [cnt, cnt, x, rgcn_weight, rgcn_root, bias2]
    if final:
        in_specs += [pl.BlockSpec((D, D), lambda i: (0, 0)),
                     pl.BlockSpec((1, D), lambda i: (0, 0))]
        args += list(wout)
    return pl.pallas_call(
        functools.partial(_combine_body, final),
        grid=(GRID,),
        in_specs=in_specs,
        out_specs=pl.BlockSpec((BN, D), lambda i: (i, 0)),
        out_shape=jax.ShapeDtypeStruct((N, D), jnp.float32),
    )(*args)


def kernel(feature, edge_index, edge_type, W_in, b_in, rgcn_weight,
           rgcn_root, rgcn_bias, W_out, b_out):
    f32 = jnp.float32
    pad = E_PAD - E
    packed = (edge_index[0] | (edge_index[1] << 14)
              | (edge_type << 28)).astype(jnp.int32)
    packed_p = jnp.concatenate(
        [packed, jnp.full((pad,), NREL << 28, jnp.int32)])
    parts4 = packed_p.reshape(NS, NPART, NBH, B)
    edge_parts = [parts4[:, e].reshape(NS * NBH, B) for e in range(NPART)]
    edges_pr = packed_p.reshape(NS * NBT, B)

    b_in2 = b_in.reshape(1, D).astype(f32)
    bias2 = rgcn_bias.reshape(1, D).astype(f32)
    wo_pad = jnp.zeros((D, D), f32).at[:, : W_out.shape[1]].set(W_out)
    bo_pad = jnp.zeros((1, D), f32).at[0, : W_out.shape[1]].set(b_out)

    conv = _make_conv_sc(False)

    def _one(r):
        return r[0] if isinstance(r, (tuple, list)) else r

    cnt = _make_count_sc()(edges_pr)
    x = _proj(feature.astype(f32), W_in, b_in2)
    xg = x.reshape(2 * N, H)
    aggs1 = [_one(conv(xg, e)) for e in edge_parts]
    x2 = _combine(False, aggs1, cnt, x, rgcn_weight, rgcn_root, bias2)
    xg2 = x2.reshape(2 * N, H)
    aggs2 = [_one(conv(xg2, e)) for e in edge_parts]
    out = _combine(True, aggs2, cnt, x2, rgcn_weight,
                   rgcn_root, bias2, wo_pad, bo_pad)
    return out[:, : W_out.shape[1]]
